# Initial kernel scaffold; baseline (speedup 1.0000x reference)
import jax
import jax.numpy as jnp
from jax.experimental import pallas as pl


def kernel(inputs, params):
    raise NotImplementedError("write your pallas kernel here")


# speedup vs baseline: 1.4807x; 1.4807x over previous
import functools

import numpy as np
import jax
import jax.numpy as jnp
from jax.experimental import pallas as pl

_K = 9
_F32 = jnp.float32
_BF16 = jnp.bfloat16
_HI = jax.lax.Precision.HIGHEST


def _conv2d(x, p, stride, pad):
    out = jax.lax.conv_general_dilated(
        x, p["w"], (stride, stride), ((pad, pad), (pad, pad)),
        dimension_numbers=("NCHW", "OIHW", "NCHW"))
    if p["b"] is not None:
        out = out + p["b"][None, :, None, None]
    return out


def _bn(x, p):
    return x * p["g"][None, :, None, None] + p["b"][None, :, None, None]


def _stem(x, p):
    x = jax.nn.gelu(_bn(_conv2d(x, p["c1a"], 1, 1), p["c1a_bn"]))
    x = jax.nn.gelu(_bn(_conv2d(x, p["c1b"], 1, 1), p["c1b_bn"]))
    tmp = x
    x = jax.nn.gelu(_bn(_conv2d(x, p["c2a"], 2, 1), p["c2a_bn"]))
    x = jax.nn.gelu(_bn(_conv2d(x, p["c2b"], 1, 1), p["c2b_bn"]))
    tmp = _bn(_conv2d(tmp, p["res1"], 2, 0), p["norm1"])
    x = x + tmp
    out = jax.nn.gelu(_bn(_conv2d(x, p["c3a"], 2, 1), p["c3a_bn"]))
    out = jax.nn.gelu(_bn(_conv2d(out, p["c3b"], 1, 1), p["c3b_bn"]))
    x = _bn(_conv2d(x, p["res2"], 2, 0), p["norm2"])
    return out + x


def _pathify(x, p):
    x = jax.nn.gelu(_bn(_conv2d(x, p["pa"], 2, 1), p["pa_bn"]))
    return _bn(_conv2d(x, p["pb"], 1, 1), p["pb_bn"])


def _affine(conv, bnp):
    g, beta = bnp["g"], bnp["b"]
    b = conv["b"] if conv["b"] is not None else jnp.zeros_like(beta)
    return g[None, :], (b * g + beta)[None, :]


def _pool_matrix(h):
    hp = h // 2
    pm = np.zeros((hp * hp, h * h), np.float32)
    for i in range(h):
        for j in range(h):
            pm[(i // 2) * hp + (j // 2), i * h + j] = 0.25
    return jnp.asarray(pm)


def _mm(a, b):
    return jnp.dot(a.astype(_BF16), b, preferred_element_type=_F32)


def _grapher_body(has_pool, n, m, c, *refs):
    if has_pool:
        (x_ref, w1, s1, c1, wg, s2, c2, w2, s3, c3,
         wf1, s4, c4, wf2, s5, c5, rel, pm, o_ref, mo_ref, mi_ref) = refs
    else:
        (x_ref, w1, s1, c1, wg, s2, c2, w2, s3, c3,
         wf1, s4, c4, wf2, s5, c5, rel, o_ref, mo_ref, mi_ref) = refs

    X = x_ref[0]
    u = _mm(X, w1[...].astype(_BF16)) * s1[...] + c1[...]
    if has_pool:
        y = jnp.dot(pm[...], u, preferred_element_type=_F32,
                    precision=_HI)
    else:
        y = u

    xn = u / (jnp.sqrt(jnp.sum(u * u, axis=1, keepdims=True)) + 1e-12)
    yn = y / (jnp.sqrt(jnp.sum(y * y, axis=1, keepdims=True)) + 1e-12)
    sxx = jnp.sum(xn * xn, axis=1, keepdims=True)
    syy = jnp.sum(yn * yn, axis=1, keepdims=True)
    xy = jax.lax.dot_general(xn.astype(_BF16), yn.astype(_BF16),
                             (((1,), (1,)), ((), ())),
                             preferred_element_type=_F32)
    dist = sxx - 2.0 * xy + jnp.transpose(syy) + rel[...]

    mcols = jax.lax.broadcasted_iota(jnp.int32, (n, m), 1)
    wgb = wg[...].astype(_BF16)
    d = dist
    mx = jnp.full((n, 2 * c), -jnp.inf, _F32)
    for _ in range(_K):
        dmin = jnp.min(d, axis=1, keepdims=True)
        sel = d == dmin
        idxk = jnp.min(jnp.where(sel, mcols, m), axis=1, keepdims=True)
        oh = mcols == idxk
        xj = jnp.dot(oh.astype(_F32), y, preferred_element_type=_F32,
                     precision=_HI)
        feat = jnp.concatenate([u, xj - u], axis=1)
        e = _mm(feat, wgb) * s2[...] + c2[...]
        mx = jnp.maximum(mx, jax.nn.gelu(e))
        d = jnp.where(oh, jnp.float32(jnp.inf), d)

    out1 = _mm(mx, w2[...].astype(_BF16)) * s3[...] + c3[...] + X
    h = jax.nn.gelu(_mm(out1, wf1[...].astype(_BF16)) * s4[...] + c4[...])
    out2 = _mm(h, wf2[...].astype(_BF16)) * s5[...] + c5[...] + out1

    o_ref[0] = out2
    mo_ref[0] = jnp.mean(out2, axis=0, keepdims=True)
    mi_ref[0] = jnp.mean(X, axis=0, keepdims=True)


def _grapher_ffn(xnod, p, r):
    b, n, c = xnod.shape
    pg, pf = p["gr"], p["ffn"]

    s1, c1 = _affine(pg["fc1"], pg["fc1_bn"])
    s2, c2 = _affine(pg["g"], pg["g_bn"])
    s3, c3 = _affine(pg["fc2"], pg["fc2_bn"])
    s4, c4 = _affine(pf["fc1"], pf["fc1_bn"])
    s5, c5 = _affine(pf["fc2"], pf["fc2_bn"])

    rel = pg["rel"][0]
    m = rel.shape[1]
    has_pool = r > 1

    ops = [
        xnod,
        pg["fc1"]["w"][:, :, 0, 0].T, s1, c1,
        pg["g"]["w"][:, :, 0, 0].T, s2, c2,
        pg["fc2"]["w"][:, :, 0, 0].T, s3, c3,
        pf["fc1"]["w"][:, :, 0, 0].T, s4, c4,
        pf["fc2"]["w"][:, :, 0, 0].T, s5, c5,
        rel,
    ]
    if has_pool:
        h = int(round(np.sqrt(n)))
        ops.append(_pool_matrix(h))

    in_specs = [pl.BlockSpec((1, n, c), lambda i: (i, 0, 0))]
    for a in ops[1:]:
        in_specs.append(
            pl.BlockSpec(a.shape, functools.partial(lambda nd, i: (0,) * nd,
                                                    a.ndim)))

    out, mo, mi = pl.pallas_call(
        functools.partial(_grapher_body, has_pool, n, m, c),
        grid=(b,),
        in_specs=in_specs,
        out_specs=[pl.BlockSpec((1, n, c), lambda i: (i, 0, 0)),
                   pl.BlockSpec((1, 1, c), lambda i: (i, 0, 0)),
                   pl.BlockSpec((1, 1, c), lambda i: (i, 0, 0))],
        out_shape=[jax.ShapeDtypeStruct((b, n, c), _F32),
                   jax.ShapeDtypeStruct((b, 1, c), _F32),
                   jax.ShapeDtypeStruct((b, 1, c), _F32)],
    )(*ops)
    return out, mo[:, 0, :], mi[:, 0, :]


def _mm_scale_body(x_ref, w_ref, s_ref, c_ref, o_ref):
    o_ref[0] = (_mm(x_ref[0], w_ref[...].astype(_BF16)) * s_ref[...]
                + c_ref[...])


def _down(xnod, p, h):
    b, n, c = xnod.shape
    img = xnod.reshape(b, h, h, c)
    xp = jnp.pad(img, ((0, 0), (1, 1), (1, 1), (0, 0)))
    taps = [xp[:, dy:dy + h:2, dx:dx + h:2, :]
            for dy in range(3) for dx in range(3)]
    ho = h // 2
    col = jnp.stack(taps, axis=-1).reshape(b, ho * ho, c * 9)

    sc, cc = _affine(p["c"], p["bn"])
    wr = p["c"]["w"].reshape(p["c"]["w"].shape[0], c * 9)
    cout = wr.shape[0]
    no = ho * ho

    out = pl.pallas_call(
        _mm_scale_body,
        grid=(b,),
        in_specs=[pl.BlockSpec((1, no, c * 9), lambda i: (i, 0, 0)),
                  pl.BlockSpec((c * 9, cout), lambda i: (0, 0)),
                  pl.BlockSpec((1, cout), lambda i: (0, 0)),
                  pl.BlockSpec((1, cout), lambda i: (0, 0))],
        out_specs=pl.BlockSpec((1, no, cout), lambda i: (i, 0, 0)),
        out_shape=jax.ShapeDtypeStruct((b, no, cout), _F32),
    )(col, wr.T, sc, cc)
    return out


def _head_body(xg_ref, l1_ref, l2_ref, l3_ref,
               xw0, xb0, hw0, hb0, cw0, cb0,
               xw1, xb1, hw1, hb1, cw1, cb1,
               xw2, xb2, hw2, hb2, cw2, cb2,
               p1w, p1b, p2w, p2b, o_ref):
    xg = xg_ref[...]
    stages = ((l1_ref, xw0, xb0, hw0, hb0, cw0, cb0),
              (l2_ref, xw1, xb1, hw1, hb1, cw1, cb1),
              (l3_ref, xw2, xb2, hw2, hb2, cw2, cb2))
    for l_ref, xw, xb, hw, hb, cw, cb in stages:
        lx = l_ref[...]
        d = lx.shape[1]
        gx = _mm(lx, xw[...].astype(_BF16)) + xb[...]
        gh = _mm(xg, hw[...].astype(_BF16)) + hb[...]
        rg = jax.nn.sigmoid(gx[:, :d] + gh[:, :d])
        zg = jax.nn.sigmoid(gx[:, d:2 * d] + gh[:, d:2 * d])
        ng = jnp.tanh(gx[:, 2 * d:] + rg * gh[:, 2 * d:])
        hy = (1.0 - zg) * ng + zg * xg
        xg = jax.nn.gelu(_mm(hy + xg, cw[...].astype(_BF16)) + cb[...])
    hfin = jax.nn.gelu(_mm(xg, p1w[...].astype(_BF16)) + p1b[...])
    o_ref[...] = _mm(hfin, p2w[...].astype(_BF16)) + p2b[...]


def _head(xg0, l1, l2, l3, P):
    b = xg0.shape[0]
    ops = [xg0, l1, l2, l3]
    for i in range(3):
        gp = P["gru%d" % i]
        cp = P["con%d" % i]
        ops += [gp["x2h"]["w"].T, gp["x2h"]["b"][None, :],
                gp["h2h"]["w"].T, gp["h2h"]["b"][None, :],
                cp["w"].T, cp["b"][None, :]]
    ops += [P["pred1"]["w"].T, P["pred1"]["b"][None, :],
            P["pred2"]["w"].T, P["pred2"]["b"][None, :]]
    return pl.pallas_call(
        _head_body,
        out_shape=jax.ShapeDtypeStruct((b, 16), _F32),
    )(*ops)


def kernel(inputs, params):
    P = params
    x = _stem(inputs, P["stem"])
    x = _pathify(x, P["path"]) + P["pos_embed"]
    b, c, h, w = x.shape
    x0 = x.reshape(b, c, h * w).transpose(0, 2, 1)

    x1, _, xg0 = _grapher_ffn(x0, P["b0"], r=1)
    x1, l1, _ = _grapher_ffn(x1, P["b1"], r=1)
    x2 = _down(x1, P["d2"], 16)
    x2, _, _ = _grapher_ffn(x2, P["b3"], r=2)
    x2, l2, _ = _grapher_ffn(x2, P["b4"], r=2)
    x3 = _down(x2, P["d5"], 8)
    x3, _, _ = _grapher_ffn(x3, P["b6"], r=1)
    x3, l3, _ = _grapher_ffn(x3, P["b7"], r=1)

    return _head(xg0, l1, l2, l3, P)


# speedup vs baseline: 1.4915x; 1.0073x over previous
import functools

import numpy as np
import jax
import jax.numpy as jnp
from jax.experimental import pallas as pl

_K = 9
_F32 = jnp.float32
_BF16 = jnp.bfloat16
_HI = jax.lax.Precision.HIGHEST


def _conv2d(x, p, stride, pad):
    out = jax.lax.conv_general_dilated(
        x, p["w"], (stride, stride), ((pad, pad), (pad, pad)),
        dimension_numbers=("NCHW", "OIHW", "NCHW"))
    if p["b"] is not None:
        out = out + p["b"][None, :, None, None]
    return out


def _bn(x, p):
    return x * p["g"][None, :, None, None] + p["b"][None, :, None, None]


def _stem(x, p):
    x = jax.nn.gelu(_bn(_conv2d(x, p["c1a"], 1, 1), p["c1a_bn"]))
    x = jax.nn.gelu(_bn(_conv2d(x, p["c1b"], 1, 1), p["c1b_bn"]))
    tmp = x
    x = jax.nn.gelu(_bn(_conv2d(x, p["c2a"], 2, 1), p["c2a_bn"]))
    x = jax.nn.gelu(_bn(_conv2d(x, p["c2b"], 1, 1), p["c2b_bn"]))
    tmp = _bn(_conv2d(tmp, p["res1"], 2, 0), p["norm1"])
    x = x + tmp
    out = jax.nn.gelu(_bn(_conv2d(x, p["c3a"], 2, 1), p["c3a_bn"]))
    out = jax.nn.gelu(_bn(_conv2d(out, p["c3b"], 1, 1), p["c3b_bn"]))
    x = _bn(_conv2d(x, p["res2"], 2, 0), p["norm2"])
    return out + x


def _pathify(x, p):
    x = jax.nn.gelu(_bn(_conv2d(x, p["pa"], 2, 1), p["pa_bn"]))
    return _bn(_conv2d(x, p["pb"], 1, 1), p["pb_bn"])


def _affine(conv, bnp):
    g, beta = bnp["g"], bnp["b"]
    b = conv["b"] if conv["b"] is not None else jnp.zeros_like(beta)
    return g[None, :], (b * g + beta)[None, :]


def _pool_matrix(h):
    hp = h // 2
    pm = np.zeros((hp * hp, h * h), np.float32)
    for i in range(h):
        for j in range(h):
            pm[(i // 2) * hp + (j // 2), i * h + j] = 0.25
    return jnp.asarray(pm)


def _mm(a, b):
    return jnp.dot(a.astype(_BF16), b, preferred_element_type=_F32)


def _grapher_body(has_pool, n, m, c, grp, *refs):
    if has_pool:
        (x_ref, w1, s1, c1, wg, s2, c2, w2, s3, c3,
         wf1, s4, c4, wf2, s5, c5, rel, pm, o_ref, mo_ref, mi_ref) = refs
    else:
        (x_ref, w1, s1, c1, wg, s2, c2, w2, s3, c3,
         wf1, s4, c4, wf2, s5, c5, rel, o_ref, mo_ref, mi_ref) = refs
    pmr = pm if has_pool else None
    for g in range(grp):
        _grapher_one(has_pool, n, m, c, g, x_ref, w1, s1, c1, wg, s2, c2,
                     w2, s3, c3, wf1, s4, c4, wf2, s5, c5, rel, pmr,
                     o_ref, mo_ref, mi_ref)


def _grapher_one(has_pool, n, m, c, g, x_ref, w1, s1, c1, wg, s2, c2,
                 w2, s3, c3, wf1, s4, c4, wf2, s5, c5, rel, pm,
                 o_ref, mo_ref, mi_ref):
    X = x_ref[g]
    u = _mm(X, w1[...].astype(_BF16)) * s1[...] + c1[...]
    if has_pool:
        y = jnp.dot(pm[...], u, preferred_element_type=_F32,
                    precision=_HI)
    else:
        y = u

    xn = u / (jnp.sqrt(jnp.sum(u * u, axis=1, keepdims=True)) + 1e-12)
    yn = y / (jnp.sqrt(jnp.sum(y * y, axis=1, keepdims=True)) + 1e-12)
    sxx = jnp.sum(xn * xn, axis=1, keepdims=True)
    syy = jnp.sum(yn * yn, axis=1, keepdims=True)
    xy = jax.lax.dot_general(xn.astype(_BF16), yn.astype(_BF16),
                             (((1,), (1,)), ((), ())),
                             preferred_element_type=_F32)
    dist = sxx - 2.0 * xy + jnp.transpose(syy) + rel[...]

    mcols = jax.lax.broadcasted_iota(jnp.int32, (n, m), 1)
    d = dist
    ohs = []
    for _ in range(_K):
        dmin = jnp.min(d, axis=1, keepdims=True)
        sel = d == dmin
        idxk = jnp.min(jnp.where(sel, mcols, m), axis=1, keepdims=True)
        oh = mcols == idxk
        ohs.append(oh)
        d = jnp.where(oh, jnp.float32(jnp.inf), d)
    ohb = jnp.concatenate(ohs, axis=0).astype(_BF16)

    y1 = y.astype(_BF16)
    r1 = y - y1.astype(_F32)
    y2 = r1.astype(_BF16)
    y3 = (r1 - y2.astype(_F32)).astype(_BF16)
    xj = (jnp.dot(ohb, y1, preferred_element_type=_F32)
          + jnp.dot(ohb, y2, preferred_element_type=_F32)
          + jnp.dot(ohb, y3, preferred_element_type=_F32))

    wgb = wg[...].astype(_BF16)
    mx = jnp.full((n, 2 * c), -jnp.inf, _F32)
    for k in range(_K):
        xjk = xj[k * n:(k + 1) * n]
        feat = jnp.concatenate([u, xjk - u], axis=1)
        e = _mm(feat, wgb) * s2[...] + c2[...]
        mx = jnp.maximum(mx, jax.nn.gelu(e))

    out1 = _mm(mx, w2[...].astype(_BF16)) * s3[...] + c3[...] + X
    h = jax.nn.gelu(_mm(out1, wf1[...].astype(_BF16)) * s4[...] + c4[...])
    out2 = _mm(h, wf2[...].astype(_BF16)) * s5[...] + c5[...] + out1

    o_ref[g] = out2
    mo_ref[g] = jnp.mean(out2, axis=0, keepdims=True)
    mi_ref[g] = jnp.mean(X, axis=0, keepdims=True)


def _grapher_ffn(xnod, p, r, grp=1):
    b, n, c = xnod.shape
    pg, pf = p["gr"], p["ffn"]

    s1, c1 = _affine(pg["fc1"], pg["fc1_bn"])
    s2, c2 = _affine(pg["g"], pg["g_bn"])
    s3, c3 = _affine(pg["fc2"], pg["fc2_bn"])
    s4, c4 = _affine(pf["fc1"], pf["fc1_bn"])
    s5, c5 = _affine(pf["fc2"], pf["fc2_bn"])

    rel = pg["rel"][0]
    m = rel.shape[1]
    has_pool = r > 1

    ops = [
        xnod,
        pg["fc1"]["w"][:, :, 0, 0].T, s1, c1,
        pg["g"]["w"][:, :, 0, 0].T, s2, c2,
        pg["fc2"]["w"][:, :, 0, 0].T, s3, c3,
        pf["fc1"]["w"][:, :, 0, 0].T, s4, c4,
        pf["fc2"]["w"][:, :, 0, 0].T, s5, c5,
        rel,
    ]
    if has_pool:
        h = int(round(np.sqrt(n)))
        ops.append(_pool_matrix(h))

    in_specs = [pl.BlockSpec((grp, n, c), lambda i: (i, 0, 0))]
    for a in ops[1:]:
        in_specs.append(
            pl.BlockSpec(a.shape, functools.partial(lambda nd, i: (0,) * nd,
                                                    a.ndim)))

    out, mo, mi = pl.pallas_call(
        functools.partial(_grapher_body, has_pool, n, m, c, grp),
        grid=(b // grp,),
        in_specs=in_specs,
        out_specs=[pl.BlockSpec((grp, n, c), lambda i: (i, 0, 0)),
                   pl.BlockSpec((grp, 1, c), lambda i: (i, 0, 0)),
                   pl.BlockSpec((grp, 1, c), lambda i: (i, 0, 0))],
        out_shape=[jax.ShapeDtypeStruct((b, n, c), _F32),
                   jax.ShapeDtypeStruct((b, 1, c), _F32),
                   jax.ShapeDtypeStruct((b, 1, c), _F32)],
    )(*ops)
    return out, mo[:, 0, :], mi[:, 0, :]


def _mm_scale_body(x_ref, w_ref, s_ref, c_ref, o_ref):
    o_ref[0] = (_mm(x_ref[0], w_ref[...].astype(_BF16)) * s_ref[...]
                + c_ref[...])


def _down(xnod, p, h):
    b, n, c = xnod.shape
    img = xnod.reshape(b, h, h, c)
    xp = jnp.pad(img, ((0, 0), (1, 1), (1, 1), (0, 0)))
    taps = [xp[:, dy:dy + h:2, dx:dx + h:2, :]
            for dy in range(3) for dx in range(3)]
    ho = h // 2
    col = jnp.stack(taps, axis=-1).reshape(b, ho * ho, c * 9)

    sc, cc = _affine(p["c"], p["bn"])
    wr = p["c"]["w"].reshape(p["c"]["w"].shape[0], c * 9)
    cout = wr.shape[0]
    no = ho * ho

    out = pl.pallas_call(
        _mm_scale_body,
        grid=(b,),
        in_specs=[pl.BlockSpec((1, no, c * 9), lambda i: (i, 0, 0)),
                  pl.BlockSpec((c * 9, cout), lambda i: (0, 0)),
                  pl.BlockSpec((1, cout), lambda i: (0, 0)),
                  pl.BlockSpec((1, cout), lambda i: (0, 0))],
        out_specs=pl.BlockSpec((1, no, cout), lambda i: (i, 0, 0)),
        out_shape=jax.ShapeDtypeStruct((b, no, cout), _F32),
    )(col, wr.T, sc, cc)
    return out


def _head_body(xg_ref, l1_ref, l2_ref, l3_ref,
               xw0, xb0, hw0, hb0, cw0, cb0,
               xw1, xb1, hw1, hb1, cw1, cb1,
               xw2, xb2, hw2, hb2, cw2, cb2,
               p1w, p1b, p2w, p2b, o_ref):
    xg = xg_ref[...]
    stages = ((l1_ref, xw0, xb0, hw0, hb0, cw0, cb0),
              (l2_ref, xw1, xb1, hw1, hb1, cw1, cb1),
              (l3_ref, xw2, xb2, hw2, hb2, cw2, cb2))
    for l_ref, xw, xb, hw, hb, cw, cb in stages:
        lx = l_ref[...]
        d = lx.shape[1]
        gx = _mm(lx, xw[...].astype(_BF16)) + xb[...]
        gh = _mm(xg, hw[...].astype(_BF16)) + hb[...]
        rg = jax.nn.sigmoid(gx[:, :d] + gh[:, :d])
        zg = jax.nn.sigmoid(gx[:, d:2 * d] + gh[:, d:2 * d])
        ng = jnp.tanh(gx[:, 2 * d:] + rg * gh[:, 2 * d:])
        hy = (1.0 - zg) * ng + zg * xg
        xg = jax.nn.gelu(_mm(hy + xg, cw[...].astype(_BF16)) + cb[...])
    hfin = jax.nn.gelu(_mm(xg, p1w[...].astype(_BF16)) + p1b[...])
    o_ref[...] = _mm(hfin, p2w[...].astype(_BF16)) + p2b[...]


def _head(xg0, l1, l2, l3, P):
    b = xg0.shape[0]
    ops = [xg0, l1, l2, l3]
    for i in range(3):
        gp = P["gru%d" % i]
        cp = P["con%d" % i]
        ops += [gp["x2h"]["w"].T, gp["x2h"]["b"][None, :],
                gp["h2h"]["w"].T, gp["h2h"]["b"][None, :],
                cp["w"].T, cp["b"][None, :]]
    ops += [P["pred1"]["w"].T, P["pred1"]["b"][None, :],
            P["pred2"]["w"].T, P["pred2"]["b"][None, :]]
    return pl.pallas_call(
        _head_body,
        out_shape=jax.ShapeDtypeStruct((b, 16), _F32),
    )(*ops)


def kernel(inputs, params):
    P = params
    x = _stem(inputs, P["stem"])
    x = _pathify(x, P["path"]) + P["pos_embed"]
    b, c, h, w = x.shape
    x0 = x.reshape(b, c, h * w).transpose(0, 2, 1)

    x1, _, xg0 = _grapher_ffn(x0, P["b0"], r=1, grp=2)
    x1, l1, _ = _grapher_ffn(x1, P["b1"], r=1, grp=2)
    x2 = _down(x1, P["d2"], 16)
    x2, _, _ = _grapher_ffn(x2, P["b3"], r=2, grp=8)
    x2, l2, _ = _grapher_ffn(x2, P["b4"], r=2, grp=8)
    x3 = _down(x2, P["d5"], 8)
    x3, _, _ = _grapher_ffn(x3, P["b6"], r=1, grp=16)
    x3, l3, _ = _grapher_ffn(x3, P["b7"], r=1, grp=16)

    return _head(xg0, l1, l2, l3, P)
